# Initial kernel scaffold; baseline (speedup 1.0000x reference)
import jax
import jax.numpy as jnp
from jax.experimental import pallas as pl


def kernel(x, feature, threshold, children_left, children_right, value):
    raise NotImplementedError("write your pallas kernel here")


# speedup vs baseline: 2672.4227x; 2672.4227x over previous
import jax
import jax.numpy as jnp
from jax import lax
from jax.experimental import pallas as pl
from jax.experimental.pallas import tpu as pltpu
from jax.experimental.pallas import tpu_sc as plsc

N_FEATURE = 256
DEPTH = 8
N_TREE = 500
N_BATCH = 4096
N_INTERNAL = 2 ** DEPTH - 1
N_LEAF = 2 ** DEPTH

NUM_CORES = 2
NUM_SUBCORES = 16
NUM_WORKERS = NUM_CORES * NUM_SUBCORES
ROWS_PER_WORKER = N_BATCH // NUM_WORKERS
LANES = 16
GROUPS = ROWS_PER_WORKER // LANES

N_TREE_PAD = 512
TCHUNK = 16
NCHUNKS = N_TREE_PAD // TCHUNK


def _forest_body(x_hbm, feat_hbm, thr_hbm, val_hbm, out_hbm,
                 x_v, feat_v, thr_v, val_v, out_v, sem_tree):
    wid = lax.axis_index("c") * NUM_SUBCORES + lax.axis_index("s")
    base = wid * ROWS_PER_WORKER

    pltpu.sync_copy(x_hbm.at[pl.ds(base, ROWS_PER_WORKER)], x_v)

    def chunk_copies(c, slot):
        ts = c * TCHUNK
        mk = pltpu.make_async_copy
        return (mk(feat_hbm.at[pl.ds(ts, TCHUNK)],
                   feat_v.at[pl.ds(slot * TCHUNK, TCHUNK)], sem_tree),
                mk(thr_hbm.at[pl.ds(ts, TCHUNK)],
                   thr_v.at[pl.ds(slot * TCHUNK, TCHUNK)], sem_tree),
                mk(val_hbm.at[pl.ds(ts, TCHUNK)],
                   val_v.at[pl.ds(slot * TCHUNK, TCHUNK)], sem_tree))

    def start_chunk(c, slot):
        for cp in chunk_copies(c, slot):
            cp.start()

    def wait_chunk(c, slot):
        for cp in chunk_copies(c, slot):
            cp.wait()

    start_chunk(0, 0)

    iota = lax.iota(jnp.int32, LANES)
    one = jnp.full((LANES,), 1, dtype=jnp.int32)
    zero = jnp.full((LANES,), 0, dtype=jnp.int32)

    def chunk_body(c, _):
        slot = lax.rem(c, 2)
        nslot = lax.rem(c + 1, 2)

        wait_chunk(c, slot)

        @pl.when(c + 1 < NCHUNKS)
        def _():
            start_chunk(c + 1, nslot)

        def tree_body(t, _):
            tloc = jnp.full((LANES,), slot * TCHUNK + t, dtype=jnp.int32)
            tout = jnp.full((LANES,), c * TCHUNK + t, dtype=jnp.int32)
            for g in range(GROUPS):
                rows = iota + (g * LANES)
                node = zero
                for _d in range(DEPTH):
                    f = plsc.load_gather(feat_v, [tloc, node])
                    th = plsc.load_gather(thr_v, [tloc, node])
                    xv = plsc.load_gather(x_v, [rows, f])
                    step = jnp.where(xv > th, one, zero)
                    node = node + node + 1 + step
                leaf = node - N_INTERNAL
                v = plsc.load_gather(val_v, [tloc, leaf])
                v = jnp.maximum(v, 0.0)
                plsc.store_scatter(out_v, [rows, tout], v)
            return ()

        lax.fori_loop(0, TCHUNK, tree_body, ())
        return ()

    lax.fori_loop(0, NCHUNKS, chunk_body, ())

    pltpu.sync_copy(out_v, out_hbm.at[pl.ds(base, ROWS_PER_WORKER)])


@jax.jit
def _forest_sc(x, feat, thr, val):
    mesh = plsc.VectorSubcoreMesh(core_axis_name="c", subcore_axis_name="s",
                                  num_cores=NUM_CORES,
                                  num_subcores=NUM_SUBCORES)
    return pl.kernel(
        _forest_body,
        out_type=jax.ShapeDtypeStruct((N_BATCH, N_TREE_PAD), jnp.float32),
        mesh=mesh,
        scratch_types=[
            pltpu.VMEM((ROWS_PER_WORKER, N_FEATURE), jnp.float32),
            pltpu.VMEM((2 * TCHUNK, N_LEAF), jnp.int32),
            pltpu.VMEM((2 * TCHUNK, N_LEAF), jnp.float32),
            pltpu.VMEM((2 * TCHUNK, N_LEAF), jnp.float32),
            pltpu.VMEM((ROWS_PER_WORKER, N_TREE_PAD), jnp.float32),
            pltpu.SemaphoreType.DMA,
        ],
        compiler_params=pltpu.CompilerParams(use_tc_tiling_on_sc=False,
                                             needs_layout_passes=False),
    )(x, feat, thr, val)


def kernel(x, feature, threshold, children_left, children_right, value):
    del children_left, children_right
    pad = ((0, N_TREE_PAD - N_TREE), (0, 0))
    feat = jnp.pad(feature[:, :N_LEAF].astype(jnp.int32), pad)
    thr = jnp.pad(threshold[:, :N_LEAF], pad)
    val = jnp.pad(value[:, N_INTERNAL:, 0], pad)
    out = _forest_sc(x, feat, thr, val)
    return out[:, :N_TREE, None]


# speedup vs baseline: 6549.0574x; 2.4506x over previous
import jax
import jax.numpy as jnp
from jax import lax
from jax.experimental import pallas as pl
from jax.experimental.pallas import tpu as pltpu
from jax.experimental.pallas import tpu_sc as plsc

N_FEATURE = 256
DEPTH = 8
N_TREE = 500
N_BATCH = 4096
N_INTERNAL = 2 ** DEPTH - 1
N_LEAF = 2 ** DEPTH

NUM_CORES = 2
NUM_SUBCORES = 16
NUM_WORKERS = NUM_CORES * NUM_SUBCORES
ROWS_PER_WORKER = N_BATCH // NUM_WORKERS
LANES = 16
GROUPS = ROWS_PER_WORKER // LANES

N_TREE_PAD = 512
TCHUNK = 16
NCHUNKS = N_TREE_PAD // TCHUNK
TWORDS = TCHUNK * N_LEAF


def _forest_body(x_hbm, feat_hbm, thr_hbm, val_hbm, out_hbm,
                 x_v, feat_v, thr_v, val_v, out_v, sem_tree):
    wid = lax.axis_index("c") * NUM_SUBCORES + lax.axis_index("s")
    base = wid * ROWS_PER_WORKER

    pltpu.sync_copy(x_hbm.at[pl.ds(base * N_FEATURE,
                                   ROWS_PER_WORKER * N_FEATURE)], x_v)

    def chunk_copies(c, slot):
        src = pl.ds(c * TWORDS, TWORDS)
        dst = pl.ds(slot * TWORDS, TWORDS)
        mk = pltpu.make_async_copy
        return (mk(feat_hbm.at[src], feat_v.at[dst], sem_tree),
                mk(thr_hbm.at[src], thr_v.at[dst], sem_tree),
                mk(val_hbm.at[src], val_v.at[dst], sem_tree))

    def start_chunk(c, slot):
        for cp in chunk_copies(c, slot):
            cp.start()

    def wait_chunk(c, slot):
        for cp in chunk_copies(c, slot):
            cp.wait()

    start_chunk(0, 0)

    iota = lax.iota(jnp.int32, LANES)
    rbase_x = [(iota + g * LANES) * N_FEATURE for g in range(GROUPS)]
    rbase_o = [(iota + g * LANES) * N_TREE_PAD for g in range(GROUPS)]

    def chunk_body(c, _):
        slot = lax.rem(c, 2)
        nslot = lax.rem(c + 1, 2)

        wait_chunk(c, slot)

        @pl.when(c + 1 < NCHUNKS)
        def _():
            start_chunk(c + 1, nslot)

        def tree_body(t, _):
            tbase = (slot * TCHUNK + t) * N_LEAF
            flat0 = jnp.full((LANES,), tbase, dtype=jnp.int32)
            go_l = jnp.full((LANES,), 1 - tbase, dtype=jnp.int32)
            go_r = jnp.full((LANES,), 2 - tbase, dtype=jnp.int32)
            tglob = jnp.full((LANES,), c * TCHUNK + t, dtype=jnp.int32)
            flats = [flat0] * GROUPS
            for _d in range(DEPTH):
                fs = [plsc.load_gather(feat_v, [flats[g]])
                      for g in range(GROUPS)]
                ths = [plsc.load_gather(thr_v, [flats[g]])
                       for g in range(GROUPS)]
                xvs = [plsc.load_gather(x_v, [rbase_x[g] + fs[g]])
                       for g in range(GROUPS)]
                flats = [flats[g] + flats[g]
                         + jnp.where(xvs[g] > ths[g], go_r, go_l)
                         for g in range(GROUPS)]
            vs = [plsc.load_gather(val_v, [flats[g] - N_INTERNAL])
                  for g in range(GROUPS)]
            for g in range(GROUPS):
                plsc.store_scatter(out_v, [rbase_o[g] + tglob],
                                   jnp.maximum(vs[g], 0.0))
            return ()

        lax.fori_loop(0, TCHUNK, tree_body, ())
        return ()

    lax.fori_loop(0, NCHUNKS, chunk_body, ())

    pltpu.sync_copy(out_v, out_hbm.at[pl.ds(base * N_TREE_PAD,
                                            ROWS_PER_WORKER * N_TREE_PAD)])


@jax.jit
def _forest_sc(x, feat, thr, val):
    mesh = plsc.VectorSubcoreMesh(core_axis_name="c", subcore_axis_name="s",
                                  num_cores=NUM_CORES,
                                  num_subcores=NUM_SUBCORES)
    return pl.kernel(
        _forest_body,
        out_type=jax.ShapeDtypeStruct((N_BATCH * N_TREE_PAD,), jnp.float32),
        mesh=mesh,
        scratch_types=[
            pltpu.VMEM((ROWS_PER_WORKER * N_FEATURE,), jnp.float32),
            pltpu.VMEM((2 * TWORDS,), jnp.int32),
            pltpu.VMEM((2 * TWORDS,), jnp.float32),
            pltpu.VMEM((2 * TWORDS,), jnp.float32),
            pltpu.VMEM((ROWS_PER_WORKER * N_TREE_PAD,), jnp.float32),
            pltpu.SemaphoreType.DMA,
        ],
        compiler_params=pltpu.CompilerParams(use_tc_tiling_on_sc=False,
                                             needs_layout_passes=False),
    )(x, feat, thr, val)


def kernel(x, feature, threshold, children_left, children_right, value):
    del children_left, children_right
    pad = ((0, N_TREE_PAD - N_TREE), (0, 0))
    feat = jnp.pad(feature[:, :N_LEAF].astype(jnp.int32), pad).reshape(-1)
    thr = jnp.pad(threshold[:, :N_LEAF], pad).reshape(-1)
    val = jnp.pad(value[:, N_INTERNAL:, 0], pad).reshape(-1)
    out = _forest_sc(x.reshape(-1), feat, thr, val)
    return out.reshape(N_BATCH, N_TREE_PAD)[:, :N_TREE, None]


# speedup vs baseline: 7293.9480x; 1.1137x over previous
import jax
import jax.numpy as jnp
from jax import lax
from jax.experimental import pallas as pl
from jax.experimental.pallas import tpu as pltpu
from jax.experimental.pallas import tpu_sc as plsc

N_FEATURE = 256
DEPTH = 8
N_TREE = 500
N_BATCH = 4096
N_INTERNAL = 2 ** DEPTH - 1
N_LEAF = 2 ** DEPTH

NUM_CORES = 2
NUM_SUBCORES = 16
NUM_WORKERS = NUM_CORES * NUM_SUBCORES
ROWS_PER_WORKER = N_BATCH // NUM_WORKERS
LANES = 16
GROUPS = ROWS_PER_WORKER // LANES

TCHUNK = 20
NCHUNKS = N_TREE // TCHUNK
TWORDS = TCHUNK * N_LEAF


def _forest_body(x_hbm, feat_hbm, thr_hbm, val_hbm, out_hbm,
                 x_v, feat_v, thr_v, val_v, out_v, sem_tree):
    wid = lax.axis_index("c") * NUM_SUBCORES + lax.axis_index("s")
    base = wid * ROWS_PER_WORKER

    pltpu.sync_copy(x_hbm.at[pl.ds(base * N_FEATURE,
                                   ROWS_PER_WORKER * N_FEATURE)], x_v)

    def chunk_copies(c, slot):
        src = pl.ds(c * TWORDS, TWORDS)
        dst = pl.ds(slot * TWORDS, TWORDS)
        mk = pltpu.make_async_copy
        return (mk(feat_hbm.at[src], feat_v.at[dst], sem_tree),
                mk(thr_hbm.at[src], thr_v.at[dst], sem_tree),
                mk(val_hbm.at[src], val_v.at[dst], sem_tree))

    def start_chunk(c, slot):
        for cp in chunk_copies(c, slot):
            cp.start()

    def wait_chunk(c, slot):
        for cp in chunk_copies(c, slot):
            cp.wait()

    start_chunk(0, 0)

    iota = lax.iota(jnp.int32, LANES)
    rbase_x = [(iota + g * LANES) * N_FEATURE for g in range(GROUPS)]
    rbase_o = [(iota + g * LANES) * N_TREE for g in range(GROUPS)]

    def chunk_body(c, _):
        slot = lax.rem(c, 2)
        nslot = lax.rem(c + 1, 2)

        wait_chunk(c, slot)

        @pl.when(c + 1 < NCHUNKS)
        def _():
            start_chunk(c + 1, nslot)

        def tree_body(t, _):
            tbase = (slot * TCHUNK + t) * N_LEAF
            flat0 = jnp.full((LANES,), tbase, dtype=jnp.int32)
            go_l = jnp.full((LANES,), 1 - tbase, dtype=jnp.int32)
            go_r = jnp.full((LANES,), 2 - tbase, dtype=jnp.int32)
            tglob = jnp.full((LANES,), c * TCHUNK + t, dtype=jnp.int32)
            flats = [flat0] * GROUPS
            for _d in range(DEPTH):
                fs = [plsc.load_gather(feat_v, [flats[g]])
                      for g in range(GROUPS)]
                ths = [plsc.load_gather(thr_v, [flats[g]])
                       for g in range(GROUPS)]
                xvs = [plsc.load_gather(x_v, [rbase_x[g] + fs[g]])
                       for g in range(GROUPS)]
                flats = [flats[g] + flats[g]
                         + jnp.where(xvs[g] > ths[g], go_r, go_l)
                         for g in range(GROUPS)]
            vs = [plsc.load_gather(val_v, [flats[g] - N_INTERNAL])
                  for g in range(GROUPS)]
            for g in range(GROUPS):
                plsc.store_scatter(out_v, [rbase_o[g] + tglob],
                                   jnp.maximum(vs[g], 0.0))
            return ()

        lax.fori_loop(0, TCHUNK, tree_body, ())
        return ()

    lax.fori_loop(0, NCHUNKS, chunk_body, ())

    pltpu.sync_copy(out_v, out_hbm.at[pl.ds(base * N_TREE,
                                            ROWS_PER_WORKER * N_TREE)])


@jax.jit
def _forest_sc(x, feat, thr, val):
    mesh = plsc.VectorSubcoreMesh(core_axis_name="c", subcore_axis_name="s",
                                  num_cores=NUM_CORES,
                                  num_subcores=NUM_SUBCORES)
    return pl.kernel(
        _forest_body,
        out_type=jax.ShapeDtypeStruct((N_BATCH * N_TREE,), jnp.float32),
        mesh=mesh,
        scratch_types=[
            pltpu.VMEM((ROWS_PER_WORKER * N_FEATURE,), jnp.float32),
            pltpu.VMEM((2 * TWORDS,), jnp.int32),
            pltpu.VMEM((2 * TWORDS,), jnp.float32),
            pltpu.VMEM((2 * TWORDS,), jnp.float32),
            pltpu.VMEM((ROWS_PER_WORKER * N_TREE,), jnp.float32),
            pltpu.SemaphoreType.DMA,
        ],
        compiler_params=pltpu.CompilerParams(use_tc_tiling_on_sc=False,
                                             needs_layout_passes=False),
    )(x, feat, thr, val)


def kernel(x, feature, threshold, children_left, children_right, value):
    del children_left, children_right
    feat = feature[:, :N_LEAF].astype(jnp.int32).reshape(-1)
    thr = threshold[:, :N_LEAF].reshape(-1)
    val = value[:, N_INTERNAL:, 0].reshape(-1)
    out = _forest_sc(x.reshape(-1), feat, thr, val)
    return out.reshape(N_BATCH, N_TREE, 1)


# speedup vs baseline: 12524.8452x; 1.7172x over previous
import jax
import jax.numpy as jnp
from jax import lax
from jax.experimental import pallas as pl
from jax.experimental.pallas import tpu as pltpu
from jax.experimental.pallas import tpu_sc as plsc

N_FEATURE = 256
DEPTH = 8
N_TREE = 500
N_BATCH = 4096
N_INTERNAL = 2 ** DEPTH - 1
N_LEAF = 2 ** DEPTH

NUM_CORES = 2
NUM_SUBCORES = 16
NUM_WORKERS = NUM_CORES * NUM_SUBCORES
ROWS_PER_WORKER = N_BATCH // NUM_WORKERS
LANES = 16
GROUPS = ROWS_PER_WORKER // LANES

TCHUNK = 20
NCHUNKS = N_TREE // TCHUNK
TWORDS = TCHUNK * N_LEAF
X_STRIDE = N_FEATURE + 1


def _forest_body(x_hbm, feat_hbm, thr_hbm, val_hbm, out_hbm,
                 x_v, feat_v, thr_v, val_v, out_v, sem_tree):
    wid = lax.axis_index("c") * NUM_SUBCORES + lax.axis_index("s")
    base = wid * ROWS_PER_WORKER

    pltpu.sync_copy(x_hbm.at[pl.ds(base * X_STRIDE,
                                   ROWS_PER_WORKER * X_STRIDE)], x_v)

    def chunk_copies(c, slot):
        src = pl.ds(c * TWORDS, TWORDS)
        dst = pl.ds(slot * TWORDS, TWORDS)
        mk = pltpu.make_async_copy
        return (mk(feat_hbm.at[src], feat_v.at[dst], sem_tree),
                mk(thr_hbm.at[src], thr_v.at[dst], sem_tree),
                mk(val_hbm.at[src], val_v.at[dst], sem_tree))

    def start_chunk(c, slot):
        for cp in chunk_copies(c, slot):
            cp.start()

    def wait_chunk(c, slot):
        for cp in chunk_copies(c, slot):
            cp.wait()

    start_chunk(0, 0)

    iota = lax.iota(jnp.int32, LANES)
    rbase_x = [(iota + g * LANES) * X_STRIDE for g in range(GROUPS)]
    rbase_o = [(iota + g * LANES) * N_TREE for g in range(GROUPS)]

    def chunk_body(c, _):
        slot = lax.rem(c, 2)
        nslot = lax.rem(c + 1, 2)

        wait_chunk(c, slot)

        @pl.when(c + 1 < NCHUNKS)
        def _():
            start_chunk(c + 1, nslot)

        def tree_body(t, _):
            tbase = (slot * TCHUNK + t) * N_LEAF
            flat0 = jnp.full((LANES,), tbase, dtype=jnp.int32)
            go_l = jnp.full((LANES,), 1 - tbase, dtype=jnp.int32)
            go_r = jnp.full((LANES,), 2 - tbase, dtype=jnp.int32)
            tglob = jnp.full((LANES,), c * TCHUNK + t, dtype=jnp.int32)
            flats = [flat0] * GROUPS
            for _d in range(DEPTH):
                fs = [plsc.load_gather(feat_v, [flats[g]])
                      for g in range(GROUPS)]
                ths = [plsc.load_gather(thr_v, [flats[g]])
                       for g in range(GROUPS)]
                xvs = [plsc.load_gather(x_v, [rbase_x[g] + fs[g]])
                       for g in range(GROUPS)]
                flats = [flats[g] + flats[g]
                         + jnp.where(xvs[g] > ths[g], go_r, go_l)
                         for g in range(GROUPS)]
            vs = [plsc.load_gather(val_v, [flats[g] - N_INTERNAL])
                  for g in range(GROUPS)]
            for g in range(GROUPS):
                plsc.store_scatter(out_v, [rbase_o[g] + tglob],
                                   jnp.maximum(vs[g], 0.0))
            return ()

        lax.fori_loop(0, TCHUNK, tree_body, ())
        return ()

    lax.fori_loop(0, NCHUNKS, chunk_body, ())

    pltpu.sync_copy(out_v, out_hbm.at[pl.ds(base * N_TREE,
                                            ROWS_PER_WORKER * N_TREE)])


@jax.jit
def _forest_sc(x, feat, thr, val):
    mesh = plsc.VectorSubcoreMesh(core_axis_name="c", subcore_axis_name="s",
                                  num_cores=NUM_CORES,
                                  num_subcores=NUM_SUBCORES)
    return pl.kernel(
        _forest_body,
        out_type=jax.ShapeDtypeStruct((N_BATCH * N_TREE,), jnp.float32),
        mesh=mesh,
        scratch_types=[
            pltpu.VMEM((ROWS_PER_WORKER * X_STRIDE,), jnp.float32),
            pltpu.VMEM((2 * TWORDS,), jnp.int32),
            pltpu.VMEM((2 * TWORDS,), jnp.float32),
            pltpu.VMEM((2 * TWORDS,), jnp.float32),
            pltpu.VMEM((ROWS_PER_WORKER * N_TREE,), jnp.float32),
            pltpu.SemaphoreType.DMA,
        ],
        compiler_params=pltpu.CompilerParams(use_tc_tiling_on_sc=False,
                                             needs_layout_passes=False),
    )(x, feat, thr, val)


def kernel(x, feature, threshold, children_left, children_right, value):
    del children_left, children_right
    feat = feature[:, :N_LEAF].astype(jnp.int32).reshape(-1)
    thr = threshold[:, :N_LEAF].reshape(-1)
    val = value[:, N_INTERNAL:, 0].reshape(-1)
    xp = jnp.pad(x, ((0, 0), (0, 1)))
    out = _forest_sc(xp.reshape(-1), feat, thr, val)
    return out.reshape(N_BATCH, N_TREE, 1)


# speedup vs baseline: 13121.5637x; 1.0476x over previous
import jax
import jax.numpy as jnp
from jax import lax
from jax.experimental import pallas as pl
from jax.experimental.pallas import tpu as pltpu
from jax.experimental.pallas import tpu_sc as plsc

N_FEATURE = 256
DEPTH = 8
N_TREE = 500
N_BATCH = 4096
N_INTERNAL = 2 ** DEPTH - 1
N_LEAF = 2 ** DEPTH

NUM_CORES = 2
NUM_SUBCORES = 16
NUM_WORKERS = NUM_CORES * NUM_SUBCORES
ROWS_PER_WORKER = N_BATCH // NUM_WORKERS
LANES = 16
GROUPS = ROWS_PER_WORKER // LANES

TCHUNK = 20
NCHUNKS = N_TREE // TCHUNK
TSIZE = 272
TWORDS = TCHUNK * TSIZE
VWORDS = TCHUNK * N_LEAF
X_STRIDE = N_FEATURE + 1


def _forest_body(x_hbm, feat_hbm, thr_hbm, val_hbm, out_hbm,
                 x_v, feat_v, thr_v, val_v, out_v, sem_tree):
    wid = lax.axis_index("c") * NUM_SUBCORES + lax.axis_index("s")
    base = wid * ROWS_PER_WORKER

    pltpu.sync_copy(x_hbm.at[pl.ds(base * X_STRIDE,
                                   ROWS_PER_WORKER * X_STRIDE)], x_v)

    def chunk_copies(c, slot):
        src = pl.ds(c * TWORDS, TWORDS)
        dst = pl.ds(slot * TWORDS, TWORDS)
        vsrc = pl.ds(c * VWORDS, VWORDS)
        vdst = pl.ds(slot * VWORDS, VWORDS)
        mk = pltpu.make_async_copy
        return (mk(feat_hbm.at[src], feat_v.at[dst], sem_tree),
                mk(thr_hbm.at[src], thr_v.at[dst], sem_tree),
                mk(val_hbm.at[vsrc], val_v.at[vdst], sem_tree))

    def start_chunk(c, slot):
        for cp in chunk_copies(c, slot):
            cp.start()

    def wait_chunk(c, slot):
        for cp in chunk_copies(c, slot):
            cp.wait()

    start_chunk(0, 0)

    iota = lax.iota(jnp.int32, LANES)
    rbase_x = [(iota + g * LANES) * X_STRIDE for g in range(GROUPS)]
    rbase_o = [(iota + g * LANES) * N_TREE for g in range(GROUPS)]

    def chunk_body(c, _):
        slot = lax.rem(c, 2)
        nslot = lax.rem(c + 1, 2)

        wait_chunk(c, slot)

        @pl.when(c + 1 < NCHUNKS)
        def _():
            start_chunk(c + 1, nslot)

        def take(vec, idx):
            return vec.at[idx].get(mode="promise_in_bounds")

        def tree_body(t, _):
            tb = (slot * TCHUNK + t) * TSIZE
            feat_a = feat_v[pl.ds(tb, LANES)]
            thr_a = thr_v[pl.ds(tb, LANES)]
            feat_b = feat_v[pl.ds(tb + LANES, LANES)]
            thr_b = thr_v[pl.ds(tb + LANES, LANES)]
            tb1 = jnp.full((LANES,), tb + 1, dtype=jnp.int32)
            go_l = jnp.full((LANES,), 1, dtype=jnp.int32)
            go_r = jnp.full((LANES,), 2, dtype=jnp.int32)
            vb = jnp.full((LANES,), (slot * TCHUNK + t) * N_LEAF - N_INTERNAL,
                          dtype=jnp.int32)
            tglob = jnp.full((LANES,), c * TCHUNK + t, dtype=jnp.int32)
            zero = jnp.full((LANES,), 0, dtype=jnp.int32)

            nodes = [zero] * GROUPS

            def descend(nodes, fs, ths):
                xvs = [plsc.load_gather(x_v, [rbase_x[g] + fs[g]])
                       for g in range(GROUPS)]
                return [nodes[g] + nodes[g]
                        + jnp.where(xvs[g] > ths[g], go_r, go_l)
                        for g in range(GROUPS)]

            for _d in range(4):
                fs = [take(feat_a, nodes[g]) for g in range(GROUPS)]
                ths = [take(thr_a, nodes[g]) for g in range(GROUPS)]
                nodes = descend(nodes, fs, ths)
            lidx = [nodes[g] - 15 for g in range(GROUPS)]
            fs = [take(feat_b, lidx[g]) for g in range(GROUPS)]
            ths = [take(thr_b, lidx[g]) for g in range(GROUPS)]
            nodes = descend(nodes, fs, ths)
            for _d in range(5, DEPTH):
                idxs = [nodes[g] + tb1 for g in range(GROUPS)]
                fs = [plsc.load_gather(feat_v, [idxs[g]])
                      for g in range(GROUPS)]
                ths = [plsc.load_gather(thr_v, [idxs[g]])
                       for g in range(GROUPS)]
                nodes = descend(nodes, fs, ths)
            vs = [plsc.load_gather(val_v, [nodes[g] + vb])
                  for g in range(GROUPS)]
            for g in range(GROUPS):
                plsc.store_scatter(out_v, [rbase_o[g] + tglob],
                                   jnp.maximum(vs[g], 0.0))
            return ()

        lax.fori_loop(0, TCHUNK, tree_body, ())
        return ()

    lax.fori_loop(0, NCHUNKS, chunk_body, ())

    pltpu.sync_copy(out_v, out_hbm.at[pl.ds(base * N_TREE,
                                            ROWS_PER_WORKER * N_TREE)])


@jax.jit
def _forest_sc(x, feat, thr, val):
    mesh = plsc.VectorSubcoreMesh(core_axis_name="c", subcore_axis_name="s",
                                  num_cores=NUM_CORES,
                                  num_subcores=NUM_SUBCORES)
    return pl.kernel(
        _forest_body,
        out_type=jax.ShapeDtypeStruct((N_BATCH * N_TREE,), jnp.float32),
        mesh=mesh,
        scratch_types=[
            pltpu.VMEM((ROWS_PER_WORKER * X_STRIDE,), jnp.float32),
            pltpu.VMEM((2 * TWORDS,), jnp.int32),
            pltpu.VMEM((2 * TWORDS,), jnp.float32),
            pltpu.VMEM((2 * VWORDS,), jnp.float32),
            pltpu.VMEM((ROWS_PER_WORKER * N_TREE,), jnp.float32),
            pltpu.SemaphoreType.DMA,
        ],
        compiler_params=pltpu.CompilerParams(use_tc_tiling_on_sc=False,
                                             needs_layout_passes=False),
    )(x, feat, thr, val)


def kernel(x, feature, threshold, children_left, children_right, value):
    del children_left, children_right
    def retile(a):
        z1 = jnp.zeros((N_TREE, 1), a.dtype)
        z15 = jnp.zeros((N_TREE, 15), a.dtype)
        return jnp.concatenate(
            [a[:, :15], z1, a[:, 15:N_LEAF], z15], axis=1).reshape(-1)

    feat = retile(feature.astype(jnp.int32))
    thr = retile(threshold)
    val = value[:, N_INTERNAL:, 0].reshape(-1)
    xp = jnp.pad(x, ((0, 0), (0, 1)))
    out = _forest_sc(xp.reshape(-1), feat, thr, val)
    return out.reshape(N_BATCH, N_TREE, 1)


# speedup vs baseline: 14065.9106x; 1.0720x over previous
import jax
import jax.numpy as jnp
from jax import lax
from jax.experimental import pallas as pl
from jax.experimental.pallas import tpu as pltpu
from jax.experimental.pallas import tpu_sc as plsc

N_FEATURE = 256
DEPTH = 8
N_TREE = 500
N_BATCH = 4096
N_INTERNAL = 2 ** DEPTH - 1
N_LEAF = 2 ** DEPTH

NUM_CORES = 2
NUM_SUBCORES = 16
NUM_WORKERS = NUM_CORES * NUM_SUBCORES
ROWS_PER_WORKER = N_BATCH // NUM_WORKERS
LANES = 16
GROUPS = ROWS_PER_WORKER // LANES

TCHUNK = 20
NCHUNKS = N_TREE // TCHUNK
TSIZE = 272
TWORDS = TCHUNK * TSIZE
VWORDS = TCHUNK * N_LEAF
X_STRIDE = N_FEATURE + 1


def _forest_body(x_hbm, feat_hbm, thr_hbm, val_hbm, out_hbm,
                 x_v, feat_v, thr_v, val_v, out_v, sem_tree):
    wid = lax.axis_index("c") * NUM_SUBCORES + lax.axis_index("s")
    base = wid * ROWS_PER_WORKER

    pltpu.sync_copy(x_hbm.at[pl.ds(base * X_STRIDE,
                                   ROWS_PER_WORKER * X_STRIDE)], x_v)

    def chunk_copies(c, slot):
        src = pl.ds(c * TWORDS, TWORDS)
        dst = pl.ds(slot * TWORDS, TWORDS)
        vsrc = pl.ds(c * VWORDS, VWORDS)
        vdst = pl.ds(slot * VWORDS, VWORDS)
        mk = pltpu.make_async_copy
        return (mk(feat_hbm.at[src], feat_v.at[dst], sem_tree),
                mk(thr_hbm.at[src], thr_v.at[dst], sem_tree),
                mk(val_hbm.at[vsrc], val_v.at[vdst], sem_tree))

    def start_chunk(c, slot):
        for cp in chunk_copies(c, slot):
            cp.start()

    def wait_chunk(c, slot):
        for cp in chunk_copies(c, slot):
            cp.wait()

    start_chunk(0, 0)

    iota = lax.iota(jnp.int32, LANES)
    rbase_x = [(iota + g * LANES) * X_STRIDE for g in range(GROUPS)]
    rbase_o = [(iota + g * LANES) * N_TREE for g in range(GROUPS)]

    def chunk_body(c, _):
        slot = lax.rem(c, 2)
        nslot = lax.rem(c + 1, 2)

        wait_chunk(c, slot)

        @pl.when(c + 1 < NCHUNKS)
        def _():
            start_chunk(c + 1, nslot)

        def take(vec, idx):
            return vec.at[idx].get(mode="promise_in_bounds")

        def tree_body(t):
            tb = (slot * TCHUNK + t) * TSIZE
            feat_a = feat_v[pl.ds(tb, LANES)]
            thr_a = thr_v[pl.ds(tb, LANES)]
            feat_b = feat_v[pl.ds(tb + LANES, LANES)]
            thr_b = thr_v[pl.ds(tb + LANES, LANES)]
            tb1 = jnp.full((LANES,), tb + 1, dtype=jnp.int32)
            go_l = jnp.full((LANES,), 1, dtype=jnp.int32)
            go_r = jnp.full((LANES,), 2, dtype=jnp.int32)
            vb = jnp.full((LANES,), (slot * TCHUNK + t) * N_LEAF - N_INTERNAL,
                          dtype=jnp.int32)
            tglob = jnp.full((LANES,), c * TCHUNK + t, dtype=jnp.int32)
            zero = jnp.full((LANES,), 0, dtype=jnp.int32)

            nodes = [zero] * GROUPS

            def descend(nodes, fs, ths):
                xvs = [plsc.load_gather(x_v, [rbase_x[g] + fs[g]])
                       for g in range(GROUPS)]
                return [nodes[g] + nodes[g]
                        + jnp.where(xvs[g] > ths[g], go_r, go_l)
                        for g in range(GROUPS)]

            for _d in range(4):
                fs = [take(feat_a, nodes[g]) for g in range(GROUPS)]
                ths = [take(thr_a, nodes[g]) for g in range(GROUPS)]
                nodes = descend(nodes, fs, ths)
            lidx = [nodes[g] - 15 for g in range(GROUPS)]
            fs = [take(feat_b, lidx[g]) for g in range(GROUPS)]
            ths = [take(thr_b, lidx[g]) for g in range(GROUPS)]
            nodes = descend(nodes, fs, ths)
            for _d in range(5, DEPTH):
                idxs = [nodes[g] + tb1 for g in range(GROUPS)]
                fs = [plsc.load_gather(feat_v, [idxs[g]])
                      for g in range(GROUPS)]
                ths = [plsc.load_gather(thr_v, [idxs[g]])
                       for g in range(GROUPS)]
                nodes = descend(nodes, fs, ths)
            vs = [plsc.load_gather(val_v, [nodes[g] + vb])
                  for g in range(GROUPS)]
            for g in range(GROUPS):
                plsc.store_scatter(out_v, [rbase_o[g] + tglob],
                                   jnp.maximum(vs[g], 0.0))

        plsc.parallel_loop(0, TCHUNK, unroll=2)(tree_body)
        return ()

    lax.fori_loop(0, NCHUNKS, chunk_body, ())

    pltpu.sync_copy(out_v, out_hbm.at[pl.ds(base * N_TREE,
                                            ROWS_PER_WORKER * N_TREE)])


@jax.jit
def _forest_sc(x, feat, thr, val):
    mesh = plsc.VectorSubcoreMesh(core_axis_name="c", subcore_axis_name="s",
                                  num_cores=NUM_CORES,
                                  num_subcores=NUM_SUBCORES)
    return pl.kernel(
        _forest_body,
        out_type=jax.ShapeDtypeStruct((N_BATCH * N_TREE,), jnp.float32),
        mesh=mesh,
        scratch_types=[
            pltpu.VMEM((ROWS_PER_WORKER * X_STRIDE,), jnp.float32),
            pltpu.VMEM((2 * TWORDS,), jnp.int32),
            pltpu.VMEM((2 * TWORDS,), jnp.float32),
            pltpu.VMEM((2 * VWORDS,), jnp.float32),
            pltpu.VMEM((ROWS_PER_WORKER * N_TREE,), jnp.float32),
            pltpu.SemaphoreType.DMA,
        ],
        compiler_params=pltpu.CompilerParams(use_tc_tiling_on_sc=False,
                                             needs_layout_passes=False),
    )(x, feat, thr, val)


def kernel(x, feature, threshold, children_left, children_right, value):
    del children_left, children_right
    def retile(a):
        z1 = jnp.zeros((N_TREE, 1), a.dtype)
        z15 = jnp.zeros((N_TREE, 15), a.dtype)
        return jnp.concatenate(
            [a[:, :15], z1, a[:, 15:N_LEAF], z15], axis=1).reshape(-1)

    feat = retile(feature.astype(jnp.int32))
    thr = retile(threshold)
    val = value[:, N_INTERNAL:, 0].reshape(-1)
    xp = jnp.pad(x, ((0, 0), (0, 1)))
    out = _forest_sc(xp.reshape(-1), feat, thr, val)
    return out.reshape(N_BATCH, N_TREE, 1)
